# baseline (speedup 1.0000x reference)
import jax
import jax.numpy as jnp
from jax import lax
from jax.experimental import pallas as pl
from jax.experimental.pallas import tpu as pltpu
from jax.experimental.pallas import tpu_sc as plsc

_Q = 1000
_QPAD = 1024
_BW = 4
_NBLK = _QPAD // _BW
_NVAL = _Q // _BW
_KPAD = 4096
_R = 512
_BIG = 3.0e38

_RPW = _KPAD // 32
_GPW = _RPW // 16


def _cswap(a, b):
    return jnp.minimum(a, b), jnp.maximum(a, b)


def _sort4(vals):
    a, b, c, d = vals
    a, b = _cswap(a, b)
    c, d = _cswap(c, d)
    a, c = _cswap(a, c)
    b, d = _cswap(b, d)
    b, c = _cswap(b, c)
    return a, b, c, d


def _tc_body(predT_ref, tgtT_ref, sel_ref):
    pT = predT_ref[0]
    tT = tgtT_ref[...]
    cT = (jnp.abs(pT[:, 0:1] - tT[0:1, :]) + jnp.abs(pT[:, 1:2] - tT[1:2, :])
          + jnp.abs(pT[:, 2:3] - tT[2:3, :])
          + jnp.abs(pT[:, 3:4] - tT[3:4, :]))

    bmin = jnp.min(cT.reshape(_NBLK, _BW, _R), axis=1)
    bmax = jnp.max(cT.reshape(_NBLK, _BW, _R), axis=1)
    siota = lax.broadcasted_iota(jnp.int32, (_NBLK, _R), 0)
    bmin = jnp.where(siota >= _NVAL, _BIG, bmin)
    bmax = jnp.where(siota >= _NVAL, -_BIG, bmax)

    mins = []
    for _ in range(4):
        m = jnp.min(bmin, axis=0, keepdims=True)
        s = jnp.min(jnp.where(bmin == m, siota, _NBLK), axis=0, keepdims=True)
        mins.append(s)
        bmin = jnp.where(siota == s, _BIG, bmin)
    maxs = []
    for _ in range(4):
        m = jnp.max(bmax, axis=0, keepdims=True)
        s = jnp.min(jnp.where(bmax == m, siota, _NBLK), axis=0, keepdims=True)
        maxs.append(s)
        bmax = jnp.where(siota == s, -_BIG, bmax)

    sel_ref[...] = jnp.concatenate(_sort4(mins) + _sort4(maxs), axis=0)


def _insert4(m, i, v, g, is_min):
    m1, m2, m3, m4 = m
    i1, i2, i3, i4 = i
    if is_min:
        c1, c2, c3, c4 = v < m1, v < m2, v < m3, v < m4
    else:
        c1, c2, c3, c4 = v > m1, v > m2, v > m3, v > m4
    n4 = jnp.where(c3, m3, jnp.where(c4, v, m4))
    j4 = jnp.where(c3, i3, jnp.where(c4, g, i4))
    n3 = jnp.where(c2, m2, jnp.where(c3, v, m3))
    j3 = jnp.where(c2, i2, jnp.where(c3, g, i3))
    n2 = jnp.where(c1, m1, jnp.where(c2, v, m2))
    j2 = jnp.where(c1, i1, jnp.where(c2, g, i2))
    n1 = jnp.where(c1, v, m1)
    j1 = jnp.where(c1, g, i1)
    return (n1, n2, n3, n4), (j1, j2, j3, j4)


def _sc_body(pt_hbm, tgtT_hbm, sel_hbm, out_hbm, p_v, t_v, sel_v, out_v, sem):
    del sem
    wid = lax.axis_index("s") * 2 + lax.axis_index("c")
    k0 = wid * _RPW
    pltpu.sync_copy(pt_hbm, p_v)
    pltpu.sync_copy(tgtT_hbm.at[:, pl.ds(k0, _RPW)], t_v)
    pltpu.sync_copy(sel_hbm.at[:, pl.ds(k0, _RPW)], sel_v)
    iota = lax.broadcasted_iota(jnp.int32, (16,), 0)
    prow = [jnp.full((16,), c, jnp.int32) for c in range(4)]

    def group(gi, carry):
        rvec = gi * 16 + iota
        tx = plsc.load_gather(t_v, [jnp.full((16,), 0, jnp.int32), rvec])
        ty = plsc.load_gather(t_v, [jnp.full((16,), 1, jnp.int32), rvec])
        tz = plsc.load_gather(t_v, [jnp.full((16,), 2, jnp.int32), rvec])
        tw = plsc.load_gather(t_v, [jnp.full((16,), 3, jnp.int32), rvec])
        obase = gi * 128
        mi = tuple(jnp.full((16,), _BIG, jnp.float32) for _ in range(4))
        ii = tuple(jnp.zeros((16,), jnp.int32) for _ in range(4))
        ma = tuple(jnp.full((16,), -_BIG, jnp.float32) for _ in range(4))
        ia = tuple(jnp.zeros((16,), jnp.int32) for _ in range(4))
        for side in range(2):
            for blk in range(4):
                s = side * 4 + blk
                bid = plsc.load_gather(
                    sel_v, [jnp.full((16,), s, jnp.int32), rvec])
                colbase = bid * _BW
                for l in range(_BW):
                    q = colbase + l
                    col = l * _NBLK + bid
                    px = plsc.load_gather(p_v, [prow[0], col])
                    py = plsc.load_gather(p_v, [prow[1], col])
                    pz = plsc.load_gather(p_v, [prow[2], col])
                    pw = plsc.load_gather(p_v, [prow[3], col])
                    v = (jnp.abs(px - tx) + jnp.abs(py - ty)
                         + jnp.abs(pz - tz) + jnp.abs(pw - tw))
                    if side == 0:
                        mi, ii = _insert4(mi, ii, v, q, True)
                    else:
                        ma, ia = _insert4(ma, ia, v, q, False)
        for k in range(4):
            plsc.store_scatter(out_v, [obase + iota * 8 + k], ii[k])
            plsc.store_scatter(out_v, [obase + iota * 8 + 4 + k], ia[k])
        return carry

    lax.fori_loop(0, _GPW, group, 0)
    pltpu.sync_copy(out_v, out_hbm.at[pl.ds(wid * (_RPW * 8), _RPW * 8)])


def _make_sc_select():
    return pl.kernel(
        _sc_body,
        out_type=jax.ShapeDtypeStruct((_KPAD * 8,), jnp.int32),
        mesh=plsc.VectorSubcoreMesh(core_axis_name="c", subcore_axis_name="s"),
        compiler_params=pltpu.CompilerParams(needs_layout_passes=False),
        scratch_types=[
            pltpu.VMEM((4, _QPAD), jnp.float32),
            pltpu.VMEM((8, _RPW), jnp.float32),
            pltpu.VMEM((8, _RPW), jnp.int32),
            pltpu.VMEM((_RPW * 8,), jnp.int32),
            pltpu.SemaphoreType.DMA,
        ],
    )


@jax.jit
def kernel(pred_boxes, anchors, targets):
    del anchors
    bs, num_q = pred_boxes.shape[:2]
    total_gt = bs * num_q

    predT = jnp.pad(pred_boxes, ((0, 0), (0, _QPAD - num_q), (0, 4)))
    pt = jnp.pad(jnp.transpose(pred_boxes, (0, 2, 1)),
                 ((0, 0), (0, 0), (0, _QPAD - num_q)))
    pt = (pt.reshape(bs, 4, _NBLK, _BW).transpose(0, 1, 3, 2)
          .reshape(bs, 4, _QPAD))
    tgtT = jnp.pad(targets.reshape(total_gt, 4).T,
                   ((0, 4), (0, _KPAD - total_gt)))

    tc_call = pl.pallas_call(
        _tc_body,
        grid=(_KPAD // _R,),
        in_specs=[pl.BlockSpec((1, _QPAD, 8), lambda j: (0, 0, 0)),
                  pl.BlockSpec((8, _R), lambda j: (0, j))],
        out_specs=pl.BlockSpec((8, _R), lambda j: (0, j)),
        out_shape=jax.ShapeDtypeStruct((8, _KPAD), jnp.int32),
    )
    sc_call = _make_sc_select()

    outs = []
    for b in range(bs):
        sel = tc_call(predT[b:b + 1], tgtT)
        outs.append(sc_call(pt[b], tgtT, sel))
    out = jnp.stack(outs)

    idx_i = out[:, :total_gt * 8].astype(jnp.int64)
    j_row = jnp.concatenate([jnp.arange(4), jnp.arange(4)])
    idx_j = jnp.broadcast_to(jnp.tile(j_row, total_gt),
                             (bs, total_gt * 8)).astype(jnp.int64)
    return (idx_i, idx_j)


# speedup vs baseline: 1.5027x; 1.5027x over previous
import jax
import jax.numpy as jnp
from jax import lax
from jax.experimental import pallas as pl
from jax.experimental.pallas import tpu as pltpu
from jax.experimental.pallas import tpu_sc as plsc

_Q = 1000
_QPAD = 1024
_BW = 8
_NBLK = _QPAD // _BW
_NVAL = _Q // _BW
_KPAD = 4096
_R = 512
_BIG = 3.0e38

_RPW = _KPAD // 32
_GPW = _RPW // 16


def _cswap(a, b):
    return jnp.minimum(a, b), jnp.maximum(a, b)


def _sort4(vals):
    a, b, c, d = vals
    a, b = _cswap(a, b)
    c, d = _cswap(c, d)
    a, c = _cswap(a, c)
    b, d = _cswap(b, d)
    b, c = _cswap(b, c)
    return a, b, c, d


def _tc_body(predT_ref, tgtT_ref, sel_ref):
    pT = predT_ref[0]
    tT = tgtT_ref[...]
    cT = (jnp.abs(pT[:, 0:1] - tT[0:1, :]) + jnp.abs(pT[:, 1:2] - tT[1:2, :])
          + jnp.abs(pT[:, 2:3] - tT[2:3, :])
          + jnp.abs(pT[:, 3:4] - tT[3:4, :]))

    bmin = jnp.min(cT.reshape(_NBLK, _BW, _R), axis=1)
    bmax = jnp.max(cT.reshape(_NBLK, _BW, _R), axis=1)
    siota = lax.broadcasted_iota(jnp.int32, (_NBLK, _R), 0)
    bmin = jnp.where(siota >= _NVAL, _BIG, bmin)
    bmax = jnp.where(siota >= _NVAL, -_BIG, bmax)

    mins = []
    for _ in range(4):
        m = jnp.min(bmin, axis=0, keepdims=True)
        s = jnp.min(jnp.where(bmin == m, siota, _NBLK), axis=0, keepdims=True)
        mins.append(s)
        bmin = jnp.where(siota == s, _BIG, bmin)
    maxs = []
    for _ in range(4):
        m = jnp.max(bmax, axis=0, keepdims=True)
        s = jnp.min(jnp.where(bmax == m, siota, _NBLK), axis=0, keepdims=True)
        maxs.append(s)
        bmax = jnp.where(siota == s, -_BIG, bmax)

    sel_ref[...] = jnp.concatenate(_sort4(mins) + _sort4(maxs), axis=0)


def _insert4(m, i, v, g, is_min):
    m1, m2, m3, m4 = m
    i1, i2, i3, i4 = i
    if is_min:
        c1, c2, c3, c4 = v < m1, v < m2, v < m3, v < m4
    else:
        c1, c2, c3, c4 = v > m1, v > m2, v > m3, v > m4
    n4 = jnp.where(c3, m3, jnp.where(c4, v, m4))
    j4 = jnp.where(c3, i3, jnp.where(c4, g, i4))
    n3 = jnp.where(c2, m2, jnp.where(c3, v, m3))
    j3 = jnp.where(c2, i2, jnp.where(c3, g, i3))
    n2 = jnp.where(c1, m1, jnp.where(c2, v, m2))
    j2 = jnp.where(c1, i1, jnp.where(c2, g, i2))
    n1 = jnp.where(c1, v, m1)
    j1 = jnp.where(c1, g, i1)
    return (n1, n2, n3, n4), (j1, j2, j3, j4)


def _sc_body(pt_hbm, tgtT_hbm, sel_hbm, out_hbm, p_v, t_v, sel_v, out_v, sem):
    del sem
    wid = lax.axis_index("s") * 2 + lax.axis_index("c")
    k0 = wid * _RPW
    pltpu.sync_copy(pt_hbm, p_v)
    pltpu.sync_copy(tgtT_hbm.at[:, pl.ds(k0, _RPW)], t_v)
    pltpu.sync_copy(sel_hbm.at[:, pl.ds(k0, _RPW)], sel_v)
    iota = lax.broadcasted_iota(jnp.int32, (16,), 0)
    prow = [jnp.full((16,), c, jnp.int32) for c in range(4)]

    def group(gi, carry):
        rvec = gi * 16 + iota
        tx = plsc.load_gather(t_v, [jnp.full((16,), 0, jnp.int32), rvec])
        ty = plsc.load_gather(t_v, [jnp.full((16,), 1, jnp.int32), rvec])
        tz = plsc.load_gather(t_v, [jnp.full((16,), 2, jnp.int32), rvec])
        tw = plsc.load_gather(t_v, [jnp.full((16,), 3, jnp.int32), rvec])
        obase = gi * 128
        mi = tuple(jnp.full((16,), _BIG, jnp.float32) for _ in range(4))
        ii = tuple(jnp.zeros((16,), jnp.int32) for _ in range(4))
        ma = tuple(jnp.full((16,), -_BIG, jnp.float32) for _ in range(4))
        ia = tuple(jnp.zeros((16,), jnp.int32) for _ in range(4))
        for side in range(2):
            for blk in range(4):
                s = side * 4 + blk
                bid = plsc.load_gather(
                    sel_v, [jnp.full((16,), s, jnp.int32), rvec])
                colbase = bid * _BW
                for l in range(_BW):
                    q = colbase + l
                    col = l * _NBLK + bid
                    px = plsc.load_gather(p_v, [prow[0], col])
                    py = plsc.load_gather(p_v, [prow[1], col])
                    pz = plsc.load_gather(p_v, [prow[2], col])
                    pw = plsc.load_gather(p_v, [prow[3], col])
                    v = (jnp.abs(px - tx) + jnp.abs(py - ty)
                         + jnp.abs(pz - tz) + jnp.abs(pw - tw))
                    if side == 0:
                        mi, ii = _insert4(mi, ii, v, q, True)
                    else:
                        ma, ia = _insert4(ma, ia, v, q, False)
        for k in range(4):
            plsc.store_scatter(out_v, [obase + iota * 8 + k], ii[k])
            plsc.store_scatter(out_v, [obase + iota * 8 + 4 + k], ia[k])
        return carry

    lax.fori_loop(0, _GPW, group, 0)
    pltpu.sync_copy(out_v, out_hbm.at[pl.ds(wid * (_RPW * 8), _RPW * 8)])


def _make_sc_select():
    return pl.kernel(
        _sc_body,
        out_type=jax.ShapeDtypeStruct((_KPAD * 8,), jnp.int32),
        mesh=plsc.VectorSubcoreMesh(core_axis_name="c", subcore_axis_name="s"),
        compiler_params=pltpu.CompilerParams(needs_layout_passes=False),
        scratch_types=[
            pltpu.VMEM((4, _QPAD), jnp.float32),
            pltpu.VMEM((8, _RPW), jnp.float32),
            pltpu.VMEM((8, _RPW), jnp.int32),
            pltpu.VMEM((_RPW * 8,), jnp.int32),
            pltpu.SemaphoreType.DMA,
        ],
    )


@jax.jit
def kernel(pred_boxes, anchors, targets):
    del anchors
    bs, num_q = pred_boxes.shape[:2]
    total_gt = bs * num_q

    predT = jnp.pad(pred_boxes, ((0, 0), (0, _QPAD - num_q), (0, 4)))
    pt = jnp.pad(jnp.transpose(pred_boxes, (0, 2, 1)),
                 ((0, 0), (0, 0), (0, _QPAD - num_q)))
    pt = (pt.reshape(bs, 4, _NBLK, _BW).transpose(0, 1, 3, 2)
          .reshape(bs, 4, _QPAD))
    tgtT = jnp.pad(targets.reshape(total_gt, 4).T,
                   ((0, 4), (0, _KPAD - total_gt)))

    tc_call = pl.pallas_call(
        _tc_body,
        grid=(_KPAD // _R,),
        in_specs=[pl.BlockSpec((1, _QPAD, 8), lambda j: (0, 0, 0)),
                  pl.BlockSpec((8, _R), lambda j: (0, j))],
        out_specs=pl.BlockSpec((8, _R), lambda j: (0, j)),
        out_shape=jax.ShapeDtypeStruct((8, _KPAD), jnp.int32),
    )
    sc_call = _make_sc_select()

    outs = []
    for b in range(bs):
        sel = tc_call(predT[b:b + 1], tgtT)
        outs.append(sc_call(pt[b], tgtT, sel))
    out = jnp.stack(outs)

    idx_i = out[:, :total_gt * 8].astype(jnp.int64)
    j_row = jnp.concatenate([jnp.arange(4), jnp.arange(4)])
    idx_j = jnp.broadcast_to(jnp.tile(j_row, total_gt),
                             (bs, total_gt * 8)).astype(jnp.int64)
    return (idx_i, idx_j)


# speedup vs baseline: 1.5031x; 1.0003x over previous
import jax
import jax.numpy as jnp
from jax import lax
from jax.experimental import pallas as pl
from jax.experimental.pallas import tpu as pltpu
from jax.experimental.pallas import tpu_sc as plsc

_Q = 1000
_QPAD = 1024
_BW = 8
_NBLK = _QPAD // _BW
_NVAL = _Q // _BW
_KPAD = 4096
_R = 1024
_BIG = 3.0e38

_RPW = _KPAD // 32
_GPW = _RPW // 16


def _cswap(a, b):
    return jnp.minimum(a, b), jnp.maximum(a, b)


def _sort4(vals):
    a, b, c, d = vals
    a, b = _cswap(a, b)
    c, d = _cswap(c, d)
    a, c = _cswap(a, c)
    b, d = _cswap(b, d)
    b, c = _cswap(b, c)
    return a, b, c, d


def _tc_body(predT_ref, tgtT_ref, sel_ref):
    pT = predT_ref[0]
    tT = tgtT_ref[...]
    cT = (jnp.abs(pT[:, 0:1] - tT[0:1, :]) + jnp.abs(pT[:, 1:2] - tT[1:2, :])
          + jnp.abs(pT[:, 2:3] - tT[2:3, :])
          + jnp.abs(pT[:, 3:4] - tT[3:4, :]))

    bmin = jnp.min(cT.reshape(_NBLK, _BW, _R), axis=1)
    bmax = jnp.max(cT.reshape(_NBLK, _BW, _R), axis=1)
    siota = lax.broadcasted_iota(jnp.int32, (_NBLK, _R), 0)
    bmin = jnp.where(siota >= _NVAL, _BIG, bmin)
    bmax = jnp.where(siota >= _NVAL, -_BIG, bmax)

    mins = []
    for _ in range(4):
        m = jnp.min(bmin, axis=0, keepdims=True)
        s = jnp.min(jnp.where(bmin == m, siota, _NBLK), axis=0, keepdims=True)
        mins.append(s)
        bmin = jnp.where(siota == s, _BIG, bmin)
    maxs = []
    for _ in range(4):
        m = jnp.max(bmax, axis=0, keepdims=True)
        s = jnp.min(jnp.where(bmax == m, siota, _NBLK), axis=0, keepdims=True)
        maxs.append(s)
        bmax = jnp.where(siota == s, -_BIG, bmax)

    sel_ref[...] = jnp.concatenate(_sort4(mins) + _sort4(maxs), axis=0)


def _insert4(m, i, v, g, is_min):
    m1, m2, m3, m4 = m
    i1, i2, i3, i4 = i
    if is_min:
        c1, c2, c3, c4 = v < m1, v < m2, v < m3, v < m4
    else:
        c1, c2, c3, c4 = v > m1, v > m2, v > m3, v > m4
    n4 = jnp.where(c3, m3, jnp.where(c4, v, m4))
    j4 = jnp.where(c3, i3, jnp.where(c4, g, i4))
    n3 = jnp.where(c2, m2, jnp.where(c3, v, m3))
    j3 = jnp.where(c2, i2, jnp.where(c3, g, i3))
    n2 = jnp.where(c1, m1, jnp.where(c2, v, m2))
    j2 = jnp.where(c1, i1, jnp.where(c2, g, i2))
    n1 = jnp.where(c1, v, m1)
    j1 = jnp.where(c1, g, i1)
    return (n1, n2, n3, n4), (j1, j2, j3, j4)


def _sc_body(pt_hbm, tgtT_hbm, sel_hbm, out_hbm, p_v, t_v, sel_v, out_v, sem):
    del sem
    wid = lax.axis_index("s") * 2 + lax.axis_index("c")
    k0 = wid * _RPW
    pltpu.sync_copy(pt_hbm, p_v)
    pltpu.sync_copy(tgtT_hbm.at[:, pl.ds(k0, _RPW)], t_v)
    pltpu.sync_copy(sel_hbm.at[:, pl.ds(k0, _RPW)], sel_v)
    iota = lax.broadcasted_iota(jnp.int32, (16,), 0)
    prow = [jnp.full((16,), c, jnp.int32) for c in range(4)]

    def group(gi, carry):
        rvec = gi * 16 + iota
        tx = plsc.load_gather(t_v, [jnp.full((16,), 0, jnp.int32), rvec])
        ty = plsc.load_gather(t_v, [jnp.full((16,), 1, jnp.int32), rvec])
        tz = plsc.load_gather(t_v, [jnp.full((16,), 2, jnp.int32), rvec])
        tw = plsc.load_gather(t_v, [jnp.full((16,), 3, jnp.int32), rvec])
        obase = gi * 128
        mi = tuple(jnp.full((16,), _BIG, jnp.float32) for _ in range(4))
        ii = tuple(jnp.zeros((16,), jnp.int32) for _ in range(4))
        ma = tuple(jnp.full((16,), -_BIG, jnp.float32) for _ in range(4))
        ia = tuple(jnp.zeros((16,), jnp.int32) for _ in range(4))
        for side in range(2):
            for blk in range(4):
                s = side * 4 + blk
                bid = plsc.load_gather(
                    sel_v, [jnp.full((16,), s, jnp.int32), rvec])
                colbase = bid * _BW
                for l in range(_BW):
                    q = colbase + l
                    col = l * _NBLK + bid
                    px = plsc.load_gather(p_v, [prow[0], col])
                    py = plsc.load_gather(p_v, [prow[1], col])
                    pz = plsc.load_gather(p_v, [prow[2], col])
                    pw = plsc.load_gather(p_v, [prow[3], col])
                    v = (jnp.abs(px - tx) + jnp.abs(py - ty)
                         + jnp.abs(pz - tz) + jnp.abs(pw - tw))
                    if side == 0:
                        mi, ii = _insert4(mi, ii, v, q, True)
                    else:
                        ma, ia = _insert4(ma, ia, v, q, False)
        for k in range(4):
            plsc.store_scatter(out_v, [obase + iota * 8 + k], ii[k])
            plsc.store_scatter(out_v, [obase + iota * 8 + 4 + k], ia[k])
        return carry

    lax.fori_loop(0, _GPW, group, 0)
    pltpu.sync_copy(out_v, out_hbm.at[pl.ds(wid * (_RPW * 8), _RPW * 8)])


def _make_sc_select():
    return pl.kernel(
        _sc_body,
        out_type=jax.ShapeDtypeStruct((_KPAD * 8,), jnp.int32),
        mesh=plsc.VectorSubcoreMesh(core_axis_name="c", subcore_axis_name="s"),
        compiler_params=pltpu.CompilerParams(needs_layout_passes=False),
        scratch_types=[
            pltpu.VMEM((4, _QPAD), jnp.float32),
            pltpu.VMEM((8, _RPW), jnp.float32),
            pltpu.VMEM((8, _RPW), jnp.int32),
            pltpu.VMEM((_RPW * 8,), jnp.int32),
            pltpu.SemaphoreType.DMA,
        ],
    )


@jax.jit
def kernel(pred_boxes, anchors, targets):
    del anchors
    bs, num_q = pred_boxes.shape[:2]
    total_gt = bs * num_q

    predT = jnp.pad(pred_boxes, ((0, 0), (0, _QPAD - num_q), (0, 4)))
    pt = jnp.pad(jnp.transpose(pred_boxes, (0, 2, 1)),
                 ((0, 0), (0, 0), (0, _QPAD - num_q)))
    pt = (pt.reshape(bs, 4, _NBLK, _BW).transpose(0, 1, 3, 2)
          .reshape(bs, 4, _QPAD))
    tgtT = jnp.pad(targets.reshape(total_gt, 4).T,
                   ((0, 4), (0, _KPAD - total_gt)))

    tc_call = pl.pallas_call(
        _tc_body,
        grid=(_KPAD // _R,),
        in_specs=[pl.BlockSpec((1, _QPAD, 8), lambda j: (0, 0, 0)),
                  pl.BlockSpec((8, _R), lambda j: (0, j))],
        out_specs=pl.BlockSpec((8, _R), lambda j: (0, j)),
        out_shape=jax.ShapeDtypeStruct((8, _KPAD), jnp.int32),
    )
    sc_call = _make_sc_select()

    outs = []
    for b in range(bs):
        sel = tc_call(predT[b:b + 1], tgtT)
        outs.append(sc_call(pt[b], tgtT, sel))
    out = jnp.stack(outs)

    idx_i = out[:, :total_gt * 8].astype(jnp.int64)
    j_row = jnp.concatenate([jnp.arange(4), jnp.arange(4)])
    idx_j = jnp.broadcast_to(jnp.tile(j_row, total_gt),
                             (bs, total_gt * 8)).astype(jnp.int64)
    return (idx_i, idx_j)


# speedup vs baseline: 1.7719x; 1.1788x over previous
import jax
import jax.numpy as jnp
from jax import lax
from jax.experimental import pallas as pl
from jax.experimental.pallas import tpu as pltpu
from jax.experimental.pallas import tpu_sc as plsc

_Q = 1000
_QPAD = 1024
_BW = 8
_NBLK = _QPAD // _BW
_NVAL = _Q // _BW
_KPAD = 4096
_R = 1024
_BIG = 3.0e38

_RPW = _KPAD // 32
_GPW = _RPW // 16


def _cswap(a, b):
    return jnp.minimum(a, b), jnp.maximum(a, b)


def _sort4(vals):
    a, b, c, d = vals
    a, b = _cswap(a, b)
    c, d = _cswap(c, d)
    a, c = _cswap(a, c)
    b, d = _cswap(b, d)
    b, c = _cswap(b, c)
    return a, b, c, d


def _tc_body(predT_ref, tgtT_ref, sel_ref):
    pT = predT_ref[0]
    tT = tgtT_ref[...]
    cT = (jnp.abs(pT[:, 0:1] - tT[0:1, :]) + jnp.abs(pT[:, 1:2] - tT[1:2, :])
          + jnp.abs(pT[:, 2:3] - tT[2:3, :])
          + jnp.abs(pT[:, 3:4] - tT[3:4, :]))

    bmin = jnp.min(cT.reshape(_BW, _NBLK, _R), axis=0)
    bmax = jnp.max(cT.reshape(_BW, _NBLK, _R), axis=0)
    siota = lax.broadcasted_iota(jnp.int32, (_NBLK, _R), 0)
    bmin = jnp.where(siota >= _NVAL, _BIG, bmin)
    bmax = jnp.where(siota >= _NVAL, -_BIG, bmax)

    mins = []
    for _ in range(4):
        m = jnp.min(bmin, axis=0, keepdims=True)
        s = jnp.min(jnp.where(bmin == m, siota, _NBLK), axis=0, keepdims=True)
        mins.append(s)
        bmin = jnp.where(siota == s, _BIG, bmin)
    maxs = []
    for _ in range(4):
        m = jnp.max(bmax, axis=0, keepdims=True)
        s = jnp.min(jnp.where(bmax == m, siota, _NBLK), axis=0, keepdims=True)
        maxs.append(s)
        bmax = jnp.where(siota == s, -_BIG, bmax)

    sel_ref[...] = jnp.concatenate(_sort4(mins) + _sort4(maxs), axis=0)


def _insert4(m, i, v, g, is_min):
    m1, m2, m3, m4 = m
    i1, i2, i3, i4 = i
    if is_min:
        c1, c2, c3, c4 = v < m1, v < m2, v < m3, v < m4
    else:
        c1, c2, c3, c4 = v > m1, v > m2, v > m3, v > m4
    n4 = jnp.where(c3, m3, jnp.where(c4, v, m4))
    j4 = jnp.where(c3, i3, jnp.where(c4, g, i4))
    n3 = jnp.where(c2, m2, jnp.where(c3, v, m3))
    j3 = jnp.where(c2, i2, jnp.where(c3, g, i3))
    n2 = jnp.where(c1, m1, jnp.where(c2, v, m2))
    j2 = jnp.where(c1, i1, jnp.where(c2, g, i2))
    n1 = jnp.where(c1, v, m1)
    j1 = jnp.where(c1, g, i1)
    return (n1, n2, n3, n4), (j1, j2, j3, j4)


def _sc_body(pt_hbm, tgtT_hbm, sel_hbm, out_hbm, p_v, t_v, sel_v, out_v, sem):
    del sem
    wid = lax.axis_index("s") * 2 + lax.axis_index("c")
    k0 = wid * _RPW
    pltpu.sync_copy(pt_hbm, p_v)
    pltpu.sync_copy(tgtT_hbm.at[:, pl.ds(k0, _RPW)], t_v)
    pltpu.sync_copy(sel_hbm.at[:, pl.ds(k0, _RPW)], sel_v)
    iota = lax.broadcasted_iota(jnp.int32, (16,), 0)
    prow = [jnp.full((16,), c, jnp.int32) for c in range(4)]

    def group(gi, carry):
        rvec = gi * 16 + iota
        tx = plsc.load_gather(t_v, [jnp.full((16,), 0, jnp.int32), rvec])
        ty = plsc.load_gather(t_v, [jnp.full((16,), 1, jnp.int32), rvec])
        tz = plsc.load_gather(t_v, [jnp.full((16,), 2, jnp.int32), rvec])
        tw = plsc.load_gather(t_v, [jnp.full((16,), 3, jnp.int32), rvec])
        obase = gi * 128
        mi = tuple(jnp.full((16,), _BIG, jnp.float32) for _ in range(4))
        ii = tuple(jnp.zeros((16,), jnp.int32) for _ in range(4))
        ma = tuple(jnp.full((16,), -_BIG, jnp.float32) for _ in range(4))
        ia = tuple(jnp.zeros((16,), jnp.int32) for _ in range(4))
        for side in range(2):
            for blk in range(4):
                s = side * 4 + blk
                bid = plsc.load_gather(
                    sel_v, [jnp.full((16,), s, jnp.int32), rvec])
                colbase = bid * _BW
                for l in range(_BW):
                    q = colbase + l
                    col = l * _NBLK + bid
                    px = plsc.load_gather(p_v, [prow[0], col])
                    py = plsc.load_gather(p_v, [prow[1], col])
                    pz = plsc.load_gather(p_v, [prow[2], col])
                    pw = plsc.load_gather(p_v, [prow[3], col])
                    v = (jnp.abs(px - tx) + jnp.abs(py - ty)
                         + jnp.abs(pz - tz) + jnp.abs(pw - tw))
                    if side == 0:
                        mi, ii = _insert4(mi, ii, v, q, True)
                    else:
                        ma, ia = _insert4(ma, ia, v, q, False)
        for k in range(4):
            plsc.store_scatter(out_v, [obase + iota * 8 + k], ii[k])
            plsc.store_scatter(out_v, [obase + iota * 8 + 4 + k], ia[k])
        return carry

    lax.fori_loop(0, _GPW, group, 0)
    pltpu.sync_copy(out_v, out_hbm.at[pl.ds(wid * (_RPW * 8), _RPW * 8)])


def _make_sc_select():
    return pl.kernel(
        _sc_body,
        out_type=jax.ShapeDtypeStruct((_KPAD * 8,), jnp.int32),
        mesh=plsc.VectorSubcoreMesh(core_axis_name="c", subcore_axis_name="s"),
        compiler_params=pltpu.CompilerParams(needs_layout_passes=False),
        scratch_types=[
            pltpu.VMEM((4, _QPAD), jnp.float32),
            pltpu.VMEM((8, _RPW), jnp.float32),
            pltpu.VMEM((8, _RPW), jnp.int32),
            pltpu.VMEM((_RPW * 8,), jnp.int32),
            pltpu.SemaphoreType.DMA,
        ],
    )


@jax.jit
def kernel(pred_boxes, anchors, targets):
    del anchors
    bs, num_q = pred_boxes.shape[:2]
    total_gt = bs * num_q

    predT = jnp.pad(pred_boxes, ((0, 0), (0, _QPAD - num_q), (0, 4)))
    predT = (predT.reshape(bs, _NBLK, _BW, 8).transpose(0, 2, 1, 3)
             .reshape(bs, _QPAD, 8))
    pt = jnp.pad(jnp.transpose(pred_boxes, (0, 2, 1)),
                 ((0, 0), (0, 0), (0, _QPAD - num_q)))
    pt = (pt.reshape(bs, 4, _NBLK, _BW).transpose(0, 1, 3, 2)
          .reshape(bs, 4, _QPAD))
    tgtT = jnp.pad(targets.reshape(total_gt, 4).T,
                   ((0, 4), (0, _KPAD - total_gt)))

    tc_call = pl.pallas_call(
        _tc_body,
        grid=(_KPAD // _R,),
        in_specs=[pl.BlockSpec((1, _QPAD, 8), lambda j: (0, 0, 0)),
                  pl.BlockSpec((8, _R), lambda j: (0, j))],
        out_specs=pl.BlockSpec((8, _R), lambda j: (0, j)),
        out_shape=jax.ShapeDtypeStruct((8, _KPAD), jnp.int32),
    )
    sc_call = _make_sc_select()

    outs = []
    for b in range(bs):
        sel = tc_call(predT[b:b + 1], tgtT)
        outs.append(sc_call(pt[b], tgtT, sel))
    out = jnp.stack(outs)

    idx_i = out[:, :total_gt * 8].astype(jnp.int64)
    j_row = jnp.concatenate([jnp.arange(4), jnp.arange(4)])
    idx_j = jnp.broadcast_to(jnp.tile(j_row, total_gt),
                             (bs, total_gt * 8)).astype(jnp.int64)
    return (idx_i, idx_j)


# speedup vs baseline: 1.7851x; 1.0075x over previous
import jax
import jax.numpy as jnp
from jax import lax
from jax.experimental import pallas as pl
from jax.experimental.pallas import tpu as pltpu
from jax.experimental.pallas import tpu_sc as plsc

_Q = 1000
_QPAD = 1024
_BW = 8
_NBLK = _QPAD // _BW
_NVAL = _Q // _BW
_KPAD = 4096
_R = 1024
_BIG = 3.0e38

_RPW = _KPAD // 32
_GPW = _RPW // 16


def _cswap(a, b):
    return jnp.minimum(a, b), jnp.maximum(a, b)


def _sort4(vals):
    a, b, c, d = vals
    a, b = _cswap(a, b)
    c, d = _cswap(c, d)
    a, c = _cswap(a, c)
    b, d = _cswap(b, d)
    b, c = _cswap(b, c)
    return a, b, c, d


def _tc_body(predT_ref, tgtT_ref, sel_ref):
    pT = predT_ref[0]
    tT = tgtT_ref[...]
    cT = (jnp.abs(pT[:, 0:1] - tT[0:1, :]) + jnp.abs(pT[:, 1:2] - tT[1:2, :])
          + jnp.abs(pT[:, 2:3] - tT[2:3, :])
          + jnp.abs(pT[:, 3:4] - tT[3:4, :]))

    bmin = jnp.min(cT.reshape(_BW, _NBLK, _R), axis=0)
    bmax = jnp.max(cT.reshape(_BW, _NBLK, _R), axis=0)
    siota = lax.broadcasted_iota(jnp.int32, (_NBLK, _R), 0)
    bmin = jnp.where(siota >= _NVAL, _BIG, bmin)
    bmax = jnp.where(siota >= _NVAL, -_BIG, bmax)

    def colmin(x):
        y = jnp.min(x.reshape(_NBLK // 16, 16, _R), axis=0)
        return jnp.min(y, axis=0, keepdims=True)

    def colmax(x):
        y = jnp.max(x.reshape(_NBLK // 16, 16, _R), axis=0)
        return jnp.max(y, axis=0, keepdims=True)

    mins = []
    for _ in range(4):
        m = colmin(bmin)
        s = colmin(jnp.where(bmin == m, siota, _NBLK))
        mins.append(s)
        bmin = jnp.where(siota == s, _BIG, bmin)
    maxs = []
    for _ in range(4):
        m = colmax(bmax)
        s = colmin(jnp.where(bmax == m, siota, _NBLK))
        maxs.append(s)
        bmax = jnp.where(siota == s, -_BIG, bmax)

    sel_ref[...] = jnp.concatenate(_sort4(mins) + _sort4(maxs), axis=0)


def _insert4(m, i, v, g, is_min):
    m1, m2, m3, m4 = m
    i1, i2, i3, i4 = i
    if is_min:
        c1, c2, c3, c4 = v < m1, v < m2, v < m3, v < m4
    else:
        c1, c2, c3, c4 = v > m1, v > m2, v > m3, v > m4
    n4 = jnp.where(c3, m3, jnp.where(c4, v, m4))
    j4 = jnp.where(c3, i3, jnp.where(c4, g, i4))
    n3 = jnp.where(c2, m2, jnp.where(c3, v, m3))
    j3 = jnp.where(c2, i2, jnp.where(c3, g, i3))
    n2 = jnp.where(c1, m1, jnp.where(c2, v, m2))
    j2 = jnp.where(c1, i1, jnp.where(c2, g, i2))
    n1 = jnp.where(c1, v, m1)
    j1 = jnp.where(c1, g, i1)
    return (n1, n2, n3, n4), (j1, j2, j3, j4)


def _sc_body(pt_hbm, tgtT_hbm, sel_hbm, out_hbm, p_v, t_v, sel_v, out_v, sem):
    del sem
    wid = lax.axis_index("s") * 2 + lax.axis_index("c")
    k0 = wid * _RPW
    pltpu.sync_copy(pt_hbm, p_v)
    pltpu.sync_copy(tgtT_hbm.at[:, pl.ds(k0, _RPW)], t_v)
    pltpu.sync_copy(sel_hbm.at[:, pl.ds(k0, _RPW)], sel_v)
    iota = lax.broadcasted_iota(jnp.int32, (16,), 0)
    prow = [jnp.full((16,), c, jnp.int32) for c in range(4)]

    def group(gi, carry):
        rvec = gi * 16 + iota
        tx = plsc.load_gather(t_v, [jnp.full((16,), 0, jnp.int32), rvec])
        ty = plsc.load_gather(t_v, [jnp.full((16,), 1, jnp.int32), rvec])
        tz = plsc.load_gather(t_v, [jnp.full((16,), 2, jnp.int32), rvec])
        tw = plsc.load_gather(t_v, [jnp.full((16,), 3, jnp.int32), rvec])
        obase = gi * 128
        mi = tuple(jnp.full((16,), _BIG, jnp.float32) for _ in range(4))
        ii = tuple(jnp.zeros((16,), jnp.int32) for _ in range(4))
        ma = tuple(jnp.full((16,), -_BIG, jnp.float32) for _ in range(4))
        ia = tuple(jnp.zeros((16,), jnp.int32) for _ in range(4))
        for side in range(2):
            for blk in range(4):
                s = side * 4 + blk
                bid = plsc.load_gather(
                    sel_v, [jnp.full((16,), s, jnp.int32), rvec])
                colbase = bid * _BW
                for l in range(_BW):
                    q = colbase + l
                    col = l * _NBLK + bid
                    px = plsc.load_gather(p_v, [prow[0], col])
                    py = plsc.load_gather(p_v, [prow[1], col])
                    pz = plsc.load_gather(p_v, [prow[2], col])
                    pw = plsc.load_gather(p_v, [prow[3], col])
                    v = (jnp.abs(px - tx) + jnp.abs(py - ty)
                         + jnp.abs(pz - tz) + jnp.abs(pw - tw))
                    if side == 0:
                        mi, ii = _insert4(mi, ii, v, q, True)
                    else:
                        ma, ia = _insert4(ma, ia, v, q, False)
        for k in range(4):
            plsc.store_scatter(out_v, [obase + iota * 8 + k], ii[k])
            plsc.store_scatter(out_v, [obase + iota * 8 + 4 + k], ia[k])
        return carry

    lax.fori_loop(0, _GPW, group, 0)
    pltpu.sync_copy(out_v, out_hbm.at[pl.ds(wid * (_RPW * 8), _RPW * 8)])


def _make_sc_select():
    return pl.kernel(
        _sc_body,
        out_type=jax.ShapeDtypeStruct((_KPAD * 8,), jnp.int32),
        mesh=plsc.VectorSubcoreMesh(core_axis_name="c", subcore_axis_name="s"),
        compiler_params=pltpu.CompilerParams(needs_layout_passes=False),
        scratch_types=[
            pltpu.VMEM((4, _QPAD), jnp.float32),
            pltpu.VMEM((8, _RPW), jnp.float32),
            pltpu.VMEM((8, _RPW), jnp.int32),
            pltpu.VMEM((_RPW * 8,), jnp.int32),
            pltpu.SemaphoreType.DMA,
        ],
    )


@jax.jit
def kernel(pred_boxes, anchors, targets):
    del anchors
    bs, num_q = pred_boxes.shape[:2]
    total_gt = bs * num_q

    predT = jnp.pad(pred_boxes, ((0, 0), (0, _QPAD - num_q), (0, 4)))
    predT = (predT.reshape(bs, _NBLK, _BW, 8).transpose(0, 2, 1, 3)
             .reshape(bs, _QPAD, 8))
    pt = jnp.pad(jnp.transpose(pred_boxes, (0, 2, 1)),
                 ((0, 0), (0, 0), (0, _QPAD - num_q)))
    pt = (pt.reshape(bs, 4, _NBLK, _BW).transpose(0, 1, 3, 2)
          .reshape(bs, 4, _QPAD))
    tgtT = jnp.pad(targets.reshape(total_gt, 4).T,
                   ((0, 4), (0, _KPAD - total_gt)))

    tc_call = pl.pallas_call(
        _tc_body,
        grid=(_KPAD // _R,),
        in_specs=[pl.BlockSpec((1, _QPAD, 8), lambda j: (0, 0, 0)),
                  pl.BlockSpec((8, _R), lambda j: (0, j))],
        out_specs=pl.BlockSpec((8, _R), lambda j: (0, j)),
        out_shape=jax.ShapeDtypeStruct((8, _KPAD), jnp.int32),
    )
    sc_call = _make_sc_select()

    outs = []
    for b in range(bs):
        sel = tc_call(predT[b:b + 1], tgtT)
        outs.append(sc_call(pt[b], tgtT, sel))
    out = jnp.stack(outs)

    idx_i = out[:, :total_gt * 8].astype(jnp.int64)
    j_row = jnp.concatenate([jnp.arange(4), jnp.arange(4)])
    idx_j = jnp.broadcast_to(jnp.tile(j_row, total_gt),
                             (bs, total_gt * 8)).astype(jnp.int64)
    return (idx_i, idx_j)
